# baseline (speedup 1.0000x reference)
import functools

import jax
import jax.numpy as jnp
from jax import lax
from jax.experimental import pallas as pl
from jax.experimental.pallas import tpu as pltpu
from jax.experimental.pallas import tpu_sc as plsc

B = 16384
CTX = 6
D = 64
L = 16
NC = 2
NS = 16
NW = NC * NS
RPW = B // NW
C = 64
NCHUNK = RPW // C
FPC = C * CTX
GPC = FPC // L
XROW = FPC // 128
OPW = RPW * CTX
IR_W = OPW // 128

_mesh = plsc.VectorSubcoreMesh(
    core_axis_name="c", subcore_axis_name="s", num_cores=NC, num_subcores=NS
)


@functools.partial(
    pl.kernel,
    out_type=jax.ShapeDtypeStruct((B * CTX,), jnp.float32),
    mesh=_mesh,
    scratch_types=[
        pltpu.VMEM((C,), jnp.int32),
        pltpu.VMEM((IR_W, 128), jnp.int32),
        pltpu.VMEM((RPW,), jnp.int32),
        pltpu.VMEM((OPW,), jnp.int32),
        pltpu.VMEM((C, 128), jnp.float32),
        pltpu.VMEM((FPC, 128), jnp.float32),
        pltpu.VMEM((OPW,), jnp.int32),
        pltpu.VMEM((OPW,), jnp.float32),
        pltpu.SemaphoreType.DMA,
    ],
    compiler_params=pltpu.CompilerParams(needs_layout_passes=False),
)
def _w2v_sc(cidx_hbm, xidx_hbm, cpar_hbm, xpar_hbm, mask_hbm,
            ctable_hbm, xtable_hbm, out_hbm,
            cidx_v, xidx_v, cpar_v, xpar_v, crows_v, xrows_v, mask_v,
            outb_v, sem):
    wid = lax.axis_index("s") * NC + lax.axis_index("c")
    obase = wid * OPW

    pltpu.sync_copy(mask_hbm.at[pl.ds(obase, OPW)], mask_v)
    pltpu.sync_copy(cpar_hbm.at[pl.ds(wid * RPW, RPW)], cpar_v)
    pltpu.sync_copy(xpar_hbm.at[pl.ds(obase, OPW)], xpar_v)
    pltpu.sync_copy(xidx_hbm.at[pl.ds(wid * IR_W, IR_W)], xidx_v)

    lanes = lax.iota(jnp.int32, L)

    for chunk in range(NCHUNK):
        cbase = chunk * FPC
        pltpu.sync_copy(cidx_hbm.at[pl.ds(wid * RPW + chunk * C, C)], cidx_v)

        handles = [pltpu.async_copy(ctable_hbm.at[cidx_v], crows_v, sem)]
        for j in range(XROW):
            handles.append(
                pltpu.async_copy(
                    xtable_hbm.at[xidx_v.at[chunk * XROW + j]],
                    xrows_v.at[pl.ds(j * 128, 128)],
                    sem,
                )
            )
        for h in handles:
            h.wait()

        def grp_body(g, _):
            fl = g * L + lanes
            rows = fl // CTX
            coff = plsc.load_gather(cpar_v, [chunk * C + rows])
            xoff = xpar_v[pl.ds(cbase + g * L, L)]
            acc = jnp.zeros((L,), jnp.float32)
            for d in range(D):
                cval = plsc.load_gather(crows_v, [rows, coff + d])
                xval = plsc.load_gather(xrows_v, [fl, xoff + d])
                acc = acc + cval * xval
            m = mask_v[pl.ds(cbase + g * L, L)]
            sig = 1.0 / (1.0 + jnp.exp(-acc))
            outb_v[pl.ds(cbase + g * L, L)] = jnp.where(m == 0, 0.0, sig)
            return 0

        lax.fori_loop(0, GPC, grp_body, 0)

    pltpu.sync_copy(outb_v, out_hbm.at[pl.ds(obase, OPW)])


def kernel(center, context, mask, center_table, context_table):
    cflat = center.reshape(-1)
    xflat = context.reshape(-1)
    cidx = cflat >> 1
    xidx = (xflat >> 1).reshape(-1, 128)
    cpar = (cflat & 1) * D
    xpar = (xflat & 1) * D
    ct2 = center_table.reshape(-1, 128)
    xt2 = context_table.reshape(-1, 128)
    out = _w2v_sc(cidx, xidx, cpar, xpar, mask.reshape(-1), ct2, xt2)
    return out.reshape(B, CTX)
